# baseline (speedup 1.0000x reference)
import functools

import jax
import jax.numpy as jnp
from jax.experimental import pallas as pl
from jax.experimental.pallas import tpu as pltpu

_NE = 8
_NC = 1000
_EC = 16
_GC = 8
_B = 128
_HW = 224
_OHW = 112
_NPIX = _OHW * _OHW
_M = _GC + _NE * _EC


_RB = 14


def _convpool_body(x_ref, w_ref, out_ref):
    xb = x_ref[0]
    xp = jnp.pad(xb, ((0, 0), (0, 2), (0, 2)))
    ph = xp.reshape(3, 113, 2, 113, 2)
    planes = []
    for ci in range(3):
        for di in range(3):
            for dj in range(3):
                p, q = di % 2, dj % 2
                ro, co = di // 2, dj // 2
                planes.append(ph[ci, :, p, :, q][:, co:co + _OHW])
    acc = jnp.zeros((_M,), dtype=jnp.float32)
    for b0 in range(0, _OHW, _RB):
        band = []
        for k in range(27):
            ro = (k % 9) // 3 // 2
            band.append(planes[k][ro + b0:ro + b0 + _RB, :])
        patches = jnp.stack(band).reshape(27, _RB * _OHW)
        patches = jnp.pad(patches, ((0, 5), (0, 0)))
        conv = jnp.dot(w_ref[:], patches,
                       preferred_element_type=jnp.float32)
        acc = acc + jnp.sum(jnp.maximum(conv, 0.0), axis=1)
    out_ref[0, 0] = acc * (1.0 / _NPIX)


def _route_combine_body(pooled_ref, gwl_ref, gbl_ref, wl_ref, bl_ref,
                        out_ref, probs_ref, aux_ref):
    pooled = pooled_ref[:]
    pg = pooled[:, :_GC]
    pe = pooled[:, _GC:]
    logits = jnp.dot(pg, gwl_ref[:],
                     preferred_element_type=jnp.float32) + gbl_ref[:]
    m = jnp.max(logits, axis=1, keepdims=True)
    e = jnp.exp(logits - m)
    probs = e / jnp.sum(e, axis=1, keepdims=True)
    bw = jnp.max(probs, axis=1, keepdims=True)
    iota_e = jax.lax.broadcasted_iota(jnp.int32, (_B, _NE), 1)
    idx = jnp.min(jnp.where(probs == bw, iota_e, _NE), axis=1, keepdims=True)
    col_e = jax.lax.broadcasted_iota(jnp.int32, (_B, _NE * _EC), 1) // _EC
    z = jnp.where(col_e == idx, pe * bw, 0.0)
    onehot_w = jnp.where(iota_e == idx, bw, 0.0)
    out = (jnp.dot(z, wl_ref[:], preferred_element_type=jnp.float32)
           + jnp.dot(onehot_w, bl_ref[:], preferred_element_type=jnp.float32))
    out_ref[:] = out
    probs_ref[:] = probs
    mean_probs = jnp.mean(probs, axis=0)
    aux_ref[0, 0] = jnp.mean((mean_probs - 1.0 / _NE) ** 2)


@jax.jit
def kernel(x, gate_wc, gate_wl, gate_bl, exp_wc, exp_wl, exp_bl):
    w136 = jnp.concatenate(
        [gate_wc.reshape(_GC, 27), exp_wc.reshape(_NE * _EC, 27)], axis=0)
    w136 = jnp.pad(w136, ((0, 0), (0, 5)))

    pooled = pl.pallas_call(
        _convpool_body,
        grid=(_B,),
        in_specs=[
            pl.BlockSpec((1, 3, _HW, _HW), lambda b: (b, 0, 0, 0)),
            pl.BlockSpec((_M, 32), lambda b: (0, 0)),
        ],
        out_specs=pl.BlockSpec((1, 1, _M), lambda b: (b, 0, 0)),
        out_shape=jax.ShapeDtypeStruct((_B, 1, _M), jnp.float32),
    )(x, w136)
    pooled = pooled.reshape(_B, _M)

    out, probs, aux = pl.pallas_call(
        _route_combine_body,
        in_specs=[pl.BlockSpec(memory_space=pltpu.VMEM)] * 5,
        out_specs=[
            pl.BlockSpec(memory_space=pltpu.VMEM),
            pl.BlockSpec(memory_space=pltpu.VMEM),
            pl.BlockSpec(memory_space=pltpu.SMEM),
        ],
        out_shape=[
            jax.ShapeDtypeStruct((_B, _NC), jnp.float32),
            jax.ShapeDtypeStruct((_B, _NE), jnp.float32),
            jax.ShapeDtypeStruct((1, 1), jnp.float32),
        ],
    )(pooled, gate_wl, gate_bl.reshape(1, _NE),
      exp_wl.reshape(_NE * _EC, _NC), exp_bl)

    return out, probs, aux.reshape(())


# speedup vs baseline: 17.4049x; 17.4049x over previous
import jax
import jax.numpy as jnp
import numpy as np
from jax.experimental import pallas as pl
from jax.experimental.pallas import tpu as pltpu

_NE = 8
_NC = 1000
_EC = 16
_GC = 8
_B = 128
_HW = 224
_OHW = 112
_NPIX = _OHW * _OHW
_M = _GC + _NE * _EC
_RB = 8
_NBLK = _OHW // _RB
_KROW = 24
_K = 9 * _KROW
_MM = _M * _RB


def _sel_matrix():
    e = np.zeros((_HW, 384), dtype=np.float32)
    for dj in range(3):
        for j in range(_OHW):
            src = 2 * j + dj
            if src < _HW:
                e[src, 128 * dj + j] = 1.0
    return jnp.asarray(e)


def _banded_weights(w136_np):
    a = jnp.zeros((_RB, _M, 3, 3, _KROW), dtype=jnp.float32)
    for t in range(_RB):
        for di in range(3):
            a = a.at[t, :, :, :, 2 * t + di].set(w136_np[:, :, di, :].transpose(0, 2, 1))
    return a.reshape(_MM, _K)


def _convpool_body(x_ref, e_ref, a_ref, out_ref):
    xb = x_ref[0]
    xr = xb.reshape(3 * 240, _HW)
    p3 = jnp.dot(xr, e_ref[:], preferred_element_type=jnp.float32)
    p3 = p3.reshape(3, 240, 384)
    acc = jnp.zeros((_MM, 128), dtype=jnp.float32)
    for blk in range(_NBLK):
        rows = p3[:, 2 * _RB * blk: 2 * _RB * blk + _KROW, :]
        parts = [rows[:, :, 128 * dj: 128 * (dj + 1)] for dj in range(3)]
        bb = jnp.stack(parts, axis=0)
        bb = bb.reshape(_K, 128)
        conv = jnp.dot(a_ref[:], bb, preferred_element_type=jnp.float32)
        acc = acc + jnp.maximum(conv, 0.0)
    pooled = jnp.sum(acc.reshape(_RB, _M, 128), axis=(0, 2)) * (1.0 / _NPIX)
    out_ref[0, 0] = pooled


def _route_combine_body(pooled_ref, gwl_ref, gbl_ref, wl_ref, bl_ref,
                        out_ref, probs_ref, aux_ref):
    pooled = pooled_ref[:]
    pg = pooled[:, :_GC]
    pe = pooled[:, _GC:]
    logits = jnp.dot(pg, gwl_ref[:],
                     preferred_element_type=jnp.float32) + gbl_ref[:]
    m = jnp.max(logits, axis=1, keepdims=True)
    e = jnp.exp(logits - m)
    probs = e / jnp.sum(e, axis=1, keepdims=True)
    bw = jnp.max(probs, axis=1, keepdims=True)
    iota_e = jax.lax.broadcasted_iota(jnp.int32, (_B, _NE), 1)
    idx = jnp.min(jnp.where(probs == bw, iota_e, _NE), axis=1, keepdims=True)
    col_e = jax.lax.broadcasted_iota(jnp.int32, (_B, _NE * _EC), 1) // _EC
    z = jnp.where(col_e == idx, pe * bw, 0.0)
    onehot_w = jnp.where(iota_e == idx, bw, 0.0)
    out = (jnp.dot(z, wl_ref[:], preferred_element_type=jnp.float32)
           + jnp.dot(onehot_w, bl_ref[:], preferred_element_type=jnp.float32))
    out_ref[:] = out
    probs_ref[:] = probs
    mean_probs = jnp.mean(probs, axis=0)
    aux_ref[0, 0] = jnp.mean((mean_probs - 1.0 / _NE) ** 2)


@jax.jit
def kernel(x, gate_wc, gate_wl, gate_bl, exp_wc, exp_wl, exp_bl):
    w136 = jnp.concatenate(
        [gate_wc.reshape(_GC, 3, 3, 3), exp_wc.reshape(_NE * _EC, 3, 3, 3)],
        axis=0)
    a_mat = _banded_weights(w136)
    e_mat = _sel_matrix()
    xpad = jnp.pad(x, ((0, 0), (0, 0), (0, 16), (0, 0)))

    pooled = pl.pallas_call(
        _convpool_body,
        grid=(_B,),
        in_specs=[
            pl.BlockSpec((1, 3, 240, _HW), lambda b: (b, 0, 0, 0)),
            pl.BlockSpec((_HW, 384), lambda b: (0, 0)),
            pl.BlockSpec((_MM, _K), lambda b: (0, 0)),
        ],
        out_specs=pl.BlockSpec((1, 1, _M), lambda b: (b, 0, 0)),
        out_shape=jax.ShapeDtypeStruct((_B, 1, _M), jnp.float32),
    )(xpad, e_mat, a_mat)
    pooled = pooled.reshape(_B, _M)

    out, probs, aux = pl.pallas_call(
        _route_combine_body,
        in_specs=[pl.BlockSpec(memory_space=pltpu.VMEM)] * 5,
        out_specs=[
            pl.BlockSpec(memory_space=pltpu.VMEM),
            pl.BlockSpec(memory_space=pltpu.VMEM),
            pl.BlockSpec(memory_space=pltpu.SMEM),
        ],
        out_shape=[
            jax.ShapeDtypeStruct((_B, _NC), jnp.float32),
            jax.ShapeDtypeStruct((_B, _NE), jnp.float32),
            jax.ShapeDtypeStruct((1, 1), jnp.float32),
        ],
    )(pooled, gate_wl, gate_bl.reshape(1, _NE),
      exp_wl.reshape(_NE * _EC, _NC), exp_bl)

    return out, probs, aux.reshape(())


# speedup vs baseline: 19.1203x; 1.0986x over previous
import jax
import jax.numpy as jnp
import numpy as np
from jax.experimental import pallas as pl
from jax.experimental.pallas import tpu as pltpu

_NE = 8
_NC = 1000
_EC = 16
_GC = 8
_B = 128
_HW = 224
_OHW = 112
_NPIX = _OHW * _OHW
_M = _GC + _NE * _EC
_RB = 8
_NBLK = _OHW // _RB
_KROW = 24
_K = 9 * _KROW
_MM = _M * _RB


def _sel_matrix():
    e = np.zeros((_HW, 384), dtype=np.float32)
    for dj in range(3):
        for j in range(_OHW):
            src = 2 * j + dj
            if src < _HW:
                e[src, 128 * dj + j] = 1.0
    return jnp.asarray(e)


def _banded_weights(w136_np):
    a = jnp.zeros((_RB, _M, 3, 3, _KROW), dtype=jnp.float32)
    for t in range(_RB):
        for di in range(3):
            a = a.at[t, :, :, :, 2 * t + di].set(w136_np[:, :, di, :].transpose(0, 2, 1))
    return a.reshape(_MM, _K)


def _convpool_body(x_ref, e_ref, a_ref, out_ref):
    xb = x_ref[0]
    xr = xb.reshape(3 * _HW, _HW)
    p3 = jnp.dot(xr, e_ref[:], preferred_element_type=jnp.float32)
    p3 = p3.reshape(3, _HW, 384)
    p3 = jnp.pad(p3, ((0, 0), (0, 16), (0, 0)))
    acc = jnp.zeros((_MM, 128), dtype=jnp.float32)
    for blk in range(_NBLK):
        rows = p3[:, 2 * _RB * blk: 2 * _RB * blk + _KROW, :]
        parts = [rows[:, :, 128 * dj: 128 * (dj + 1)] for dj in range(3)]
        bb = jnp.stack(parts, axis=0)
        bb = bb.reshape(_K, 128)
        conv = jnp.dot(a_ref[:], bb, preferred_element_type=jnp.float32)
        acc = acc + jnp.maximum(conv, 0.0)
    pooled = jnp.sum(acc.reshape(_RB, _M, 128), axis=(0, 2)) * (1.0 / _NPIX)
    out_ref[0, 0] = pooled


def _route_combine_body(pooled_ref, gwl_ref, gbl_ref, wl_ref, bl_ref,
                        out_ref, probs_ref, aux_ref):
    pooled = pooled_ref[:]
    pg = pooled[:, :_GC]
    pe = pooled[:, _GC:]
    logits = jnp.dot(pg, gwl_ref[:],
                     preferred_element_type=jnp.float32) + gbl_ref[:]
    m = jnp.max(logits, axis=1, keepdims=True)
    e = jnp.exp(logits - m)
    probs = e / jnp.sum(e, axis=1, keepdims=True)
    bw = jnp.max(probs, axis=1, keepdims=True)
    iota_e = jax.lax.broadcasted_iota(jnp.int32, (_B, _NE), 1)
    idx = jnp.min(jnp.where(probs == bw, iota_e, _NE), axis=1, keepdims=True)
    col_e = jax.lax.broadcasted_iota(jnp.int32, (_B, _NE * _EC), 1) // _EC
    z = jnp.where(col_e == idx, pe * bw, 0.0)
    onehot_w = jnp.where(iota_e == idx, bw, 0.0)
    out = (jnp.dot(z, wl_ref[:], preferred_element_type=jnp.float32)
           + jnp.dot(onehot_w, bl_ref[:], preferred_element_type=jnp.float32))
    out_ref[:] = out
    probs_ref[:] = probs
    mean_probs = jnp.mean(probs, axis=0)
    aux_ref[0, 0] = jnp.mean((mean_probs - 1.0 / _NE) ** 2)


@jax.jit
def kernel(x, gate_wc, gate_wl, gate_bl, exp_wc, exp_wl, exp_bl):
    w136 = jnp.concatenate(
        [gate_wc.reshape(_GC, 3, 3, 3), exp_wc.reshape(_NE * _EC, 3, 3, 3)],
        axis=0)
    a_mat = _banded_weights(w136)
    e_mat = _sel_matrix()

    pooled = pl.pallas_call(
        _convpool_body,
        grid=(_B,),
        in_specs=[
            pl.BlockSpec((1, 3, _HW, _HW), lambda b: (b, 0, 0, 0)),
            pl.BlockSpec((_HW, 384), lambda b: (0, 0)),
            pl.BlockSpec((_MM, _K), lambda b: (0, 0)),
        ],
        out_specs=pl.BlockSpec((1, 1, _M), lambda b: (b, 0, 0)),
        out_shape=jax.ShapeDtypeStruct((_B, 1, _M), jnp.float32),
    )(x, e_mat, a_mat)
    pooled = pooled.reshape(_B, _M)

    out, probs, aux = pl.pallas_call(
        _route_combine_body,
        in_specs=[pl.BlockSpec(memory_space=pltpu.VMEM)] * 5,
        out_specs=[
            pl.BlockSpec(memory_space=pltpu.VMEM),
            pl.BlockSpec(memory_space=pltpu.VMEM),
            pl.BlockSpec(memory_space=pltpu.SMEM),
        ],
        out_shape=[
            jax.ShapeDtypeStruct((_B, _NC), jnp.float32),
            jax.ShapeDtypeStruct((_B, _NE), jnp.float32),
            jax.ShapeDtypeStruct((1, 1), jnp.float32),
        ],
    )(pooled, gate_wl, gate_bl.reshape(1, _NE),
      exp_wl.reshape(_NE * _EC, _NC), exp_bl)

    return out, probs, aux.reshape(())
